# baseline (speedup 1.0000x reference)
import jax
import jax.numpy as jnp
from jax.experimental import pallas as pl
from jax.experimental.pallas import tpu as pltpu

_CROWS = 1024
_NBUF = 4
_PF = 2


def _add_kernel(seq_len, x_hbm, e_hbm, o_hbm, xb, ev, sin, sout, semb):
    rows = x_hbm.shape[0]
    nch = rows // _CROWS
    nseg = seq_len // _CROWS

    def start_in(g):
        return pltpu.make_async_copy(
            x_hbm.at[pl.ds(g * _CROWS, _CROWS)], xb.at[g % _NBUF],
            sin.at[g % _NBUF])

    def start_out(g):
        return pltpu.make_async_copy(
            xb.at[g % _NBUF], o_hbm.at[pl.ds(g * _CROWS, _CROWS)],
            sout.at[g % _NBUF])

    def start_emb(k):
        return pltpu.make_async_copy(
            e_hbm.at[pl.ds(k * _CROWS, _CROWS)],
            ev.at[pl.ds(k * _CROWS, _CROWS)], semb.at[k])

    start_emb(0).start()
    for g in range(_PF):
        start_in(g).start()
    for k in range(1, nseg):
        start_emb(k).start()

    for g in range(nch):
        if g + _PF < nch:
            if g + _PF >= _NBUF:
                start_out(g + _PF - _NBUF).wait()
            start_in(g + _PF).start()
        start_in(g).wait()
        if g < nseg:
            start_emb(g).wait()
        b = g % _NBUF
        xb[b] = xb[b] + ev[pl.ds((g % nseg) * _CROWS, _CROWS), :]
        start_out(g).start()
    for g in range(max(0, nch - _NBUF), nch):
        start_out(g).wait()


def kernel(inputs, embeddings):
    batch, seq_len, dim = inputs.shape
    pos = embeddings[:seq_len]
    x2 = inputs.reshape(batch * seq_len, dim)
    out = pl.pallas_call(
        lambda *refs: _add_kernel(seq_len, *refs),
        in_specs=[
            pl.BlockSpec(memory_space=pl.ANY),
            pl.BlockSpec(memory_space=pl.ANY),
        ],
        out_specs=pl.BlockSpec(memory_space=pl.ANY),
        out_shape=jax.ShapeDtypeStruct(x2.shape, x2.dtype),
        scratch_shapes=[
            pltpu.VMEM((_NBUF, _CROWS, dim), jnp.float32),
            pltpu.VMEM((seq_len, dim), jnp.float32),
            pltpu.SemaphoreType.DMA((_NBUF,)),
            pltpu.SemaphoreType.DMA((_NBUF,)),
            pltpu.SemaphoreType.DMA((seq_len // _CROWS,)),
        ],
    )(x2, pos)
    return out.reshape(inputs.shape)


# speedup vs baseline: 1.0024x; 1.0024x over previous
import jax
import jax.numpy as jnp
from jax.experimental import pallas as pl


_S_BLK = 2048


def _add_kernel(x_ref, e_ref, o_ref):
    o_ref[...] = x_ref[...] + e_ref[...]


def kernel(inputs, embeddings):
    batch, seq_len, dim = inputs.shape
    pos = embeddings[:seq_len]
    ns = seq_len // _S_BLK
    return pl.pallas_call(
        _add_kernel,
        grid=(ns, batch),
        in_specs=[
            pl.BlockSpec((1, _S_BLK, dim), lambda s, b: (b, s, 0)),
            pl.BlockSpec((_S_BLK, dim), lambda s, b: (s, 0)),
        ],
        out_specs=pl.BlockSpec((1, _S_BLK, dim), lambda s, b: (b, s, 0)),
        out_shape=jax.ShapeDtypeStruct(inputs.shape, inputs.dtype),
    )(inputs, pos)
